# baseline (speedup 1.0000x reference)
import functools

import jax
import jax.numpy as jnp
from jax import lax
from jax.experimental import pallas as pl
from jax.experimental.pallas import tpu as pltpu, tpu_sc as plsc

N = 10000
D = 128
E = 320000

NC = 2
NS = 16
NW = NC * NS
CHB = 128
NCH = 80
EPAD = NW * NCH * CHB
N_ACC = 10240
ROWS_PER_TILE_Z = N_ACC // NS


def _xpw_body(x_ref, p_ref, w_ref, o_ref):
    o_ref[...] = jnp.dot(x_ref[...] * p_ref[...], w_ref[...],
                         preferred_element_type=jnp.float32)


def _xpw(x, p2, W):
    blk = 1000
    return pl.pallas_call(
        _xpw_body,
        grid=(N // blk,),
        in_specs=[
            pl.BlockSpec((blk, D), lambda i: (i, 0)),
            pl.BlockSpec((blk, 1), lambda i: (i, 0)),
            pl.BlockSpec((D, D), lambda i: (0, 0)),
        ],
        out_specs=pl.BlockSpec((blk, D), lambda i: (i, 0)),
        out_shape=jax.ShapeDtypeStruct((N, D), jnp.float32),
    )(x, p2, W)


def _sc_body(y_hbm, pk_hbm, zeros_hbm, out_hbm,
             pk_v, sch0, dch0, sch1, dch1, buf0, buf1, acc, sem0, sem1):
    c = lax.axis_index("c")
    s = lax.axis_index("s")
    w = s * NC + c

    pltpu.sync_copy(pk_hbm.at[w], pk_v)

    pltpu.sync_copy(zeros_hbm, buf0)
    for q in range(ROWS_PER_TILE_Z // CHB):
        pltpu.sync_copy(buf0, acc.at[pl.ds(s * ROWS_PER_TILE_Z + q * CHB, CHB)])
    plsc.subcore_barrier()

    def unpack_src(j, sch):
        for k in range(CHB // 16):
            v = pk_v[j, pl.ds(k * 16, 16)]
            sch[0, pl.ds(k * 16, 16)] = lax.bitwise_and(v, 0xFFFF)

    def unpack_dst(j, dch):
        for k in range(CHB // 16):
            v = pk_v[j, pl.ds(k * 16, 16)]
            dch[0, pl.ds(k * 16, 16)] = lax.shift_right_logical(v, 16)

    def gather(sch, buf, sem):
        pltpu.async_copy(y_hbm.at[sch.at[0]], buf, sem)

    def gwait(buf, sem):
        pltpu.make_async_copy(y_hbm.at[pl.ds(0, CHB)], buf, sem).wait()

    unpack_src(0, sch0)
    gather(sch0, buf0, sem0)
    unpack_src(1, sch1)
    gather(sch1, buf1, sem1)

    def body(g, carry):
        j0 = g * 2
        gwait(buf0, sem0)
        unpack_dst(j0, dch0)
        pltpu.sync_copy(buf0, acc.at[dch0.at[0]], add=True)
        unpack_src(j0 + 2, sch0)
        gather(sch0, buf0, sem0)
        gwait(buf1, sem1)
        unpack_dst(j0 + 1, dch1)
        pltpu.sync_copy(buf1, acc.at[dch1.at[0]], add=True)
        unpack_src(j0 + 3, sch1)
        gather(sch1, buf1, sem1)
        return carry

    lax.fori_loop(0, NCH // 2, body, 0)
    gwait(buf0, sem0)
    gwait(buf1, sem1)
    plsc.subcore_barrier()

    for q in range(ROWS_PER_TILE_Z // CHB):
        r0 = s * ROWS_PER_TILE_Z + q * CHB
        pltpu.sync_copy(acc.at[pl.ds(r0, CHB)], buf0)
        pltpu.sync_copy(buf0, out_hbm.at[c].at[pl.ds(r0, CHB)])


def _sc_agg(y, pk_r, zeros):
    mesh = plsc.VectorSubcoreMesh(core_axis_name="c", subcore_axis_name="s")
    k = pl.kernel(
        _sc_body,
        out_type=jax.ShapeDtypeStruct((NC, N_ACC, D), jnp.float32),
        mesh=mesh,
        scratch_types=[
            pltpu.VMEM((NCH + 2, CHB), jnp.int32),
            pltpu.VMEM((1, CHB), jnp.int32),
            pltpu.VMEM((1, CHB), jnp.int32),
            pltpu.VMEM((1, CHB), jnp.int32),
            pltpu.VMEM((1, CHB), jnp.int32),
            pltpu.VMEM((CHB, D), jnp.float32),
            pltpu.VMEM((CHB, D), jnp.float32),
            pltpu.VMEM_SHARED((N_ACC, D), jnp.float32),
            pltpu.SemaphoreType.DMA,
            pltpu.SemaphoreType.DMA,
        ],
    )
    return k(y, pk_r, zeros)


def _fin_body(a_ref, b_ref, bias_ref, o_ref):
    o_ref[...] = a_ref[...] + b_ref[...] + bias_ref[...]


def _fin(p0, p1, bias2):
    blk = 1000
    return pl.pallas_call(
        _fin_body,
        grid=(N // blk,),
        in_specs=[
            pl.BlockSpec((blk, D), lambda i: (i, 0)),
            pl.BlockSpec((blk, D), lambda i: (i, 0)),
            pl.BlockSpec((1, D), lambda i: (0, 0)),
        ],
        out_specs=pl.BlockSpec((blk, D), lambda i: (i, 0)),
        out_shape=jax.ShapeDtypeStruct((N, D), jnp.float32),
    )(p0, p1, bias2)


@jax.jit
def kernel(x, edge_index, p, W, b):
    y = _xpw(x, p.reshape(N, 1), W)

    pad = EPAD - E
    src = jnp.concatenate([edge_index[0], jnp.zeros((pad,), jnp.int32)])
    dst = jnp.concatenate([edge_index[1], jnp.full((pad,), N, jnp.int32)])
    pk = jnp.bitwise_or(src, jnp.left_shift(dst, 16)).reshape(NW, NCH, CHB)
    pk_r = jnp.concatenate(
        [pk, jnp.zeros((NW, 2, CHB), jnp.int32)], axis=1)
    zeros = jnp.zeros((CHB, D), jnp.float32)

    parts = _sc_agg(y, pk_r, zeros)
    return _fin(parts[0], parts[1], b.reshape(1, D))


# speedup vs baseline: 1.0790x; 1.0790x over previous
import functools

import jax
import jax.numpy as jnp
from jax import lax
from jax.experimental import pallas as pl
from jax.experimental.pallas import tpu as pltpu, tpu_sc as plsc

N = 10000
D = 128
E = 320000

NC = 2
NS = 16
NW = NC * NS
CHB = 128
NCH = 80
EPAD = NW * NCH * CHB
N_ACC = 10240
ROWS_PER_TILE_Z = N_ACC // NS


def _xpw_body(x_ref, p_ref, w_ref, o_ref):
    o_ref[...] = jnp.dot(x_ref[...] * p_ref[...], w_ref[...],
                         preferred_element_type=jnp.float32)


def _xpw(x, p2, W):
    blk = 1000
    return pl.pallas_call(
        _xpw_body,
        grid=(N // blk,),
        in_specs=[
            pl.BlockSpec((blk, D), lambda i: (i, 0)),
            pl.BlockSpec((blk, 1), lambda i: (i, 0)),
            pl.BlockSpec((D, D), lambda i: (0, 0)),
        ],
        out_specs=pl.BlockSpec((blk, D), lambda i: (i, 0)),
        out_shape=jax.ShapeDtypeStruct((N, D), jnp.float32),
    )(x, p2, W)


def _sc_body(y_hbm, idx_hbm, zeros_hbm, out_hbm,
             i2, buf0, buf1, acc,
             isem0, isem1, isem2, isem3, gsem0, gsem1):
    c = lax.axis_index("c")
    s = lax.axis_index("s")
    w = s * NC + c
    isems = (isem0, isem1, isem2, isem3)
    gbufs = (buf0, buf1)
    gsems = (gsem0, gsem1)

    pltpu.sync_copy(zeros_hbm, buf0)
    for q in range(ROWS_PER_TILE_Z // CHB):
        pltpu.sync_copy(buf0, acc.at[pl.ds(s * ROWS_PER_TILE_Z + q * CHB, CHB)])

    def idx_dma(j, q):
        pltpu.async_copy(idx_hbm.at[w, j], i2.at[q], isems[q])

    def idx_wait(q):
        pltpu.make_async_copy(idx_hbm.at[0, 0], i2.at[q], isems[q]).wait()

    def gather(q, b):
        pltpu.async_copy(y_hbm.at[i2.at[q].at[0]], gbufs[b], gsems[b])

    def gwait(b):
        pltpu.make_async_copy(y_hbm.at[pl.ds(0, CHB)], gbufs[b], gsems[b]).wait()

    def scatter(q, b):
        pltpu.sync_copy(gbufs[b], acc.at[i2.at[q].at[1]], add=True)

    plsc.subcore_barrier()

    for q in range(4):
        idx_dma(q, q)
    idx_wait(0)
    gather(0, 0)
    idx_wait(1)
    gather(1, 1)

    def body(g, carry):
        j = g * 4
        for pos in range(4):
            b = pos % 2
            q = pos % 4
            gwait(b)
            scatter(q, b)
            idx_dma(j + pos + 4, q)
            idx_wait((pos + 2) % 4)
            gather((pos + 2) % 4, b)
        return carry

    lax.fori_loop(0, NCH // 4, body, 0)
    gwait(0)
    gwait(1)
    idx_wait(2)
    idx_wait(3)
    plsc.subcore_barrier()

    for q in range(ROWS_PER_TILE_Z // CHB):
        r0 = s * ROWS_PER_TILE_Z + q * CHB
        pltpu.sync_copy(acc.at[pl.ds(r0, CHB)], buf0)
        pltpu.sync_copy(buf0, out_hbm.at[c].at[pl.ds(r0, CHB)])


def _sc_agg(y, idx2, zeros):
    mesh = plsc.VectorSubcoreMesh(core_axis_name="c", subcore_axis_name="s")
    k = pl.kernel(
        _sc_body,
        out_type=jax.ShapeDtypeStruct((NC, N_ACC, D), jnp.float32),
        mesh=mesh,
        scratch_types=[
            pltpu.VMEM((4, 2, CHB), jnp.int32),
            pltpu.VMEM((CHB, D), jnp.float32),
            pltpu.VMEM((CHB, D), jnp.float32),
            pltpu.VMEM_SHARED((N_ACC, D), jnp.float32),
            pltpu.SemaphoreType.DMA,
            pltpu.SemaphoreType.DMA,
            pltpu.SemaphoreType.DMA,
            pltpu.SemaphoreType.DMA,
            pltpu.SemaphoreType.DMA,
            pltpu.SemaphoreType.DMA,
        ],
    )
    return k(y, idx2, zeros)


def _fin_body(a_ref, b_ref, bias_ref, o_ref):
    o_ref[...] = a_ref[...] + b_ref[...] + bias_ref[...]


def _fin(p0, p1, bias2):
    blk = 1000
    return pl.pallas_call(
        _fin_body,
        grid=(N // blk,),
        in_specs=[
            pl.BlockSpec((blk, D), lambda i: (i, 0)),
            pl.BlockSpec((blk, D), lambda i: (i, 0)),
            pl.BlockSpec((1, D), lambda i: (0, 0)),
        ],
        out_specs=pl.BlockSpec((blk, D), lambda i: (i, 0)),
        out_shape=jax.ShapeDtypeStruct((N, D), jnp.float32),
    )(p0, p1, bias2)


@jax.jit
def kernel(x, edge_index, p, W, b):
    y = _xpw(x, p.reshape(N, 1), W)

    pad = EPAD - E
    src = jnp.concatenate([edge_index[0], jnp.zeros((pad,), jnp.int32)])
    dst = jnp.concatenate([edge_index[1], jnp.full((pad,), N, jnp.int32)])
    sd = jnp.stack([src.reshape(NW, NCH, CHB), dst.reshape(NW, NCH, CHB)],
                   axis=2)
    idx2 = jnp.concatenate(
        [sd, jnp.zeros((NW, 4, 2, CHB), jnp.int32)], axis=1)
    zeros = jnp.zeros((CHB, D), jnp.float32)

    parts = _sc_agg(y, idx2, zeros)
    return _fin(parts[0], parts[1], b.reshape(1, D))
